# baseline (speedup 1.0000x reference)
import functools

import jax
import jax.numpy as jnp
from jax import lax
from jax.experimental import pallas as pl
from jax.experimental.pallas import tpu as pltpu
from jax.experimental.pallas import tpu_sc as plsc

N = 10000
E = 640000
NUM_GRAPHS = 64

NC = 2
NS = 16
NWORK = NC * NS
NP = 10240
ROWS_PER_SUB = NP // NS
W = 80
K = 2
E_PAD = 655360
NCHUNK = E_PAD // W
CHUNKS_PER_WORKER = NCHUNK // NWORK
GPW = CHUNKS_PER_WORKER // K
RING = 4

_MESH = plsc.VectorSubcoreMesh(core_axis_name="c", subcore_axis_name="s")
_SC_PARAMS = pltpu.CompilerParams(use_tc_tiling_on_sc=False)


def _sc_seg_sum(feat, src_r, dst_r, zeros, d):
  @functools.partial(
      pl.kernel,
      out_type=jax.ShapeDtypeStruct((NC, NP, d), jnp.float32),
      mesh=_MESH,
      scratch_types=[
          pltpu.VMEM_SHARED((NP, d), jnp.float32),
          pltpu.VMEM((RING, K, W), jnp.int32),
          pltpu.VMEM((RING, K, W), jnp.int32),
          pltpu.VMEM((RING, K, W, d), jnp.float32),
          pltpu.SemaphoreType.DMA,
          pltpu.SemaphoreType.DMA,
          pltpu.SemaphoreType.DMA,
          pltpu.SemaphoreType.DMA,
          pltpu.SemaphoreType.DMA,
          pltpu.SemaphoreType.DMA,
      ],
      compiler_params=_SC_PARAMS,
  )
  def k(feat_hbm, src_hbm, dst_hbm, z_hbm, agg_out, acc, sidx, didx, rows,
        g0, g1, g2, g3, s0, s1):
    cid = lax.axis_index("c")
    sid = lax.axis_index("s")
    wid = sid * NC + cid
    r0 = sid * ROWS_PER_SUB
    wbase = wid * CHUNKS_PER_WORKER
    gsems = (g0, g1, g2, g3)
    ssems = (s0, s1)

    pltpu.sync_copy(z_hbm.at[pl.ds(r0, ROWS_PER_SUB)],
                    acc.at[pl.ds(r0, ROWS_PER_SUB)])
    plsc.subcore_barrier()

    def load(slot, row0):
      pltpu.sync_copy(src_hbm.at[pl.ds(row0, K)], sidx.at[slot])
      pltpu.sync_copy(dst_hbm.at[pl.ds(row0, K)], didx.at[slot])

    def fire_g(slot):
      for j in range(K):
        pltpu.async_copy(feat_hbm.at[sidx.at[slot, j]], rows.at[slot, j],
                         gsems[slot])

    def drain_g(slot):
      for j in range(K):
        pltpu.make_async_copy(feat_hbm.at[sidx.at[slot, j]], rows.at[slot, j],
                              gsems[slot]).wait()

    def fire_s(slot):
      for j in range(K):
        pltpu.async_copy(rows.at[slot, j], acc.at[didx.at[slot, j]],
                         ssems[slot % 2], add=True)

    def drain_s(slot):
      for j in range(K):
        pltpu.make_async_copy(rows.at[slot, j], acc.at[didx.at[slot, j]],
                              ssems[slot % 2]).wait()

    for g in (0, 1):
      load(g, wbase + g * K)
      fire_g(g)
    for g in (0, 1):
      load(g + 2, wbase + (g + 2) * K)
      fire_g(g + 2)
      drain_g(g)
      fire_s(g)

    @pl.loop(0, (GPW - 4) // 4)
    def _(u):
      for b in range(4):
        slot = (2 + b) % 4
        pslot = (slot + 2) % 4
        gcur = 4 * u + 2 + b
        drain_s(pslot)
        load(pslot, wbase + (gcur + 2) * K)
        fire_g(pslot)
        drain_g(slot)
        fire_s(slot)

    for g in (GPW - 2, GPW - 1):
      slot = g % 4
      drain_s((slot + 2) % 4)
      drain_g(slot)
      fire_s(slot)
    for g in (GPW - 2, GPW - 1):
      drain_s(g % 4)

    plsc.subcore_barrier()
    pltpu.sync_copy(acc.at[pl.ds(r0, ROWS_PER_SUB)],
                    agg_out.at[cid, pl.ds(r0, ROWS_PER_SUB)])

  return k(feat, src_r, dst_r, zeros)


def _leaky(h):
  return jnp.where(h > 0, h, 0.01 * h)


def _tc_mlp1(x, w1, b1, w2, b2):
  def body(x_ref, w1_ref, b1_ref, w2_ref, b2_ref, o_ref):
    h = jnp.dot(x_ref[...], w1_ref[...], preferred_element_type=jnp.float32)
    h = jnp.maximum(h + b1_ref[...], 0.0)
    h = jnp.dot(h, w2_ref[...], preferred_element_type=jnp.float32)
    h = _leaky(h + b2_ref[...])
    o_ref[...] = jnp.concatenate(
        [h, jnp.ones((N, 16), jnp.float32)], axis=1)

  return pl.pallas_call(
      body, out_shape=jax.ShapeDtypeStruct((N, 80), jnp.float32)
  )(x, w1, b1, w2, b2)


def _tc_mid(aggp, h1p, s1wl, s1bl, s1wr, s2wl):
  def body(aggp_ref, h1_ref, wl_ref, bl_ref, wr_ref, w2l_ref,
           h2_ref, g2_ref, inv_ref):
    deg = aggp_ref[0, :N, 64:65] + aggp_ref[1, :N, 64:65]
    inv = 1.0 / jnp.maximum(deg, 1.0)
    inv_ref[...] = inv
    agg = (aggp_ref[0, :N, :64] + aggp_ref[1, :N, :64]) * inv
    h2 = jnp.dot(agg, wl_ref[...], preferred_element_type=jnp.float32)
    h2 = h2 + bl_ref[...]
    h2 = h2 + jnp.dot(h1_ref[:, :64], wr_ref[...],
                      preferred_element_type=jnp.float32)
    h2 = _leaky(h2)
    h2_ref[...] = h2
    g2_ref[...] = jnp.dot(h2, w2l_ref[...], preferred_element_type=jnp.float32)

  return pl.pallas_call(
      body,
      out_shape=[
          jax.ShapeDtypeStruct((N, 128), jnp.float32),
          jax.ShapeDtypeStruct((N, 64), jnp.float32),
          jax.ShapeDtypeStruct((N, 1), jnp.float32),
      ],
  )(aggp, h1p, s1wl, s1bl, s1wr, s2wl)


def _tc_final(agg2p, inv, h2, s2bl, s2wr, l2w1, l2b1, l2w2, l2b2, batch_row):
  def body(agg2p_ref, inv_ref, h2_ref, bl_ref, wr_ref, w1_ref, b1_ref,
           w2_ref, b2_ref, batch_ref, o_ref):
    t = (agg2p_ref[0, :N, :] + agg2p_ref[1, :N, :]) * inv_ref[...]
    t = t + bl_ref[...]
    t = t + jnp.dot(h2_ref[...], wr_ref[...],
                    preferred_element_type=jnp.float32)
    t = _leaky(t)
    t = jnp.dot(t, w1_ref[...], preferred_element_type=jnp.float32)
    t = jnp.maximum(t + b1_ref[...], 0.0)
    t = jnp.dot(t, w2_ref[...], preferred_element_type=jnp.float32)
    t = t + b2_ref[...]
    onehot_t = (lax.broadcasted_iota(jnp.int32, (NUM_GRAPHS, N), 0)
                == batch_ref[...]).astype(jnp.float32)
    o_ref[...] = jnp.dot(onehot_t, t, preferred_element_type=jnp.float32)

  return pl.pallas_call(
      body,
      out_shape=jax.ShapeDtypeStruct((NUM_GRAPHS, 10), jnp.float32),
  )(agg2p, inv, h2, s2bl, s2wr, l2w1, l2b1, l2w2, l2b2, batch_row)


def kernel(x, edge_index, edge_weight, batch,
           l1w1, l1b1, l1w2, l1b2,
           s1wl, s1bl, s1wr,
           s2wl, s2bl, s2wr,
           l2w1, l2b1, l2w2, l2b2):
  pad_src = jnp.zeros((E_PAD - E,), jnp.int32)
  pad_dst = jnp.full((E_PAD - E,), NP - 1, jnp.int32)
  src_r = jnp.concatenate([edge_index[0], pad_src]).reshape(NCHUNK, W)
  dst_r = jnp.concatenate([edge_index[1], pad_dst]).reshape(NCHUNK, W)
  z80 = jnp.zeros((NP, 80), jnp.float32)
  z64 = jnp.zeros((NP, 64), jnp.float32)

  h1p = _tc_mlp1(x, l1w1, l1b1.reshape(1, -1), l1w2, l1b2.reshape(1, -1))
  aggp = _sc_seg_sum(h1p, src_r, dst_r, z80, 80)
  h2, g2, inv = _tc_mid(aggp, h1p, s1wl, s1bl.reshape(1, -1), s1wr, s2wl)
  agg2p = _sc_seg_sum(g2, src_r, dst_r, z64, 64)
  out = _tc_final(agg2p, inv, h2, s2bl.reshape(1, -1), s2wr,
                  l2w1, l2b1.reshape(1, -1), l2w2, l2b2.reshape(1, -1),
                  batch.reshape(1, N).astype(jnp.int32))
  return out


# speedup vs baseline: 3.4219x; 3.4219x over previous
import functools

import jax
import jax.numpy as jnp
from jax import lax
from jax.experimental import pallas as pl
from jax.experimental.pallas import tpu as pltpu
from jax.experimental.pallas import tpu_sc as plsc

N = 10000
E = 640000
NUM_GRAPHS = 64

NC = 2
NS = 16
NWORK = NC * NS
NP = 10240
ROWS_PER_SUB = NP // NS
W = 80
K = 5
NCHUNK = E // W
CHUNKS_PER_WORKER = NCHUNK // NWORK
GROUPS_PER_WORKER = CHUNKS_PER_WORKER // K

_MESH = plsc.VectorSubcoreMesh(core_axis_name="c", subcore_axis_name="s")
_SC_PARAMS = pltpu.CompilerParams(use_tc_tiling_on_sc=False)


def _sc_seg_sum(feat, src_r, dst_r, zeros, d):
  @functools.partial(
      pl.kernel,
      out_type=jax.ShapeDtypeStruct((NC, NP, d), jnp.float32),
      mesh=_MESH,
      scratch_types=[
          pltpu.VMEM_SHARED((NP, d), jnp.float32),
          pltpu.VMEM((2, K, W), jnp.int32),
          pltpu.VMEM((2, K, W), jnp.int32),
          pltpu.VMEM((2, K, W, d), jnp.float32),
          pltpu.SemaphoreType.DMA,
          pltpu.SemaphoreType.DMA,
      ],
      compiler_params=_SC_PARAMS,
  )
  def k(feat_hbm, src_hbm, dst_hbm, z_hbm, agg_out, acc, sidx, didx, rows,
        gsem0, gsem1):
    cid = lax.axis_index("c")
    sid = lax.axis_index("s")
    wid = sid * NC + cid
    r0 = sid * ROWS_PER_SUB
    wbase = wid * CHUNKS_PER_WORKER
    gsems = (gsem0, gsem1)

    pltpu.sync_copy(z_hbm.at[pl.ds(r0, ROWS_PER_SUB)],
                    acc.at[pl.ds(r0, ROWS_PER_SUB)])
    plsc.subcore_barrier()

    def load_and_fire(buf, row0):
      pltpu.sync_copy(src_hbm.at[pl.ds(row0, K)], sidx.at[buf])
      pltpu.sync_copy(dst_hbm.at[pl.ds(row0, K)], didx.at[buf])
      for j in range(K):
        pltpu.async_copy(feat_hbm.at[sidx.at[buf, j]], rows.at[buf, j],
                         gsems[buf])

    def drain_and_scatter(buf):
      for j in range(K):
        pltpu.make_async_copy(feat_hbm.at[sidx.at[buf, j]], rows.at[buf, j],
                              gsems[buf]).wait()
        pltpu.sync_copy(rows.at[buf, j], acc.at[didx.at[buf, j]], add=True)

    for buf in range(2):
      load_and_fire(buf, wbase + buf * K)

    @pl.loop(0, GROUPS_PER_WORKER // 2 - 1)
    def _(t):
      for buf in range(2):
        drain_and_scatter(buf)
        load_and_fire(buf, wbase + (2 * t + buf + 2) * K)

    for buf in range(2):
      drain_and_scatter(buf)

    plsc.subcore_barrier()
    pltpu.sync_copy(acc.at[pl.ds(r0, ROWS_PER_SUB)],
                    agg_out.at[cid, pl.ds(r0, ROWS_PER_SUB)])

  return k(feat, src_r, dst_r, zeros)


def _leaky(h):
  return jnp.where(h > 0, h, 0.01 * h)


def _tc_mlp1(x, w1, b1, w2, b2):
  def body(x_ref, w1_ref, b1_ref, w2_ref, b2_ref, o_ref):
    h = jnp.dot(x_ref[...], w1_ref[...], preferred_element_type=jnp.float32)
    h = jnp.maximum(h + b1_ref[...], 0.0)
    h = jnp.dot(h, w2_ref[...], preferred_element_type=jnp.float32)
    h = _leaky(h + b2_ref[...])
    o_ref[...] = jnp.concatenate(
        [h, jnp.ones((N, 16), jnp.float32)], axis=1)

  return pl.pallas_call(
      body, out_shape=jax.ShapeDtypeStruct((N, 80), jnp.float32)
  )(x, w1, b1, w2, b2)


def _tc_mid(aggp, h1p, s1wl, s1bl, s1wr, s2wl):
  def body(aggp_ref, h1_ref, wl_ref, bl_ref, wr_ref, w2l_ref,
           h2_ref, g2_ref, inv_ref):
    deg = aggp_ref[0, :N, 64:65] + aggp_ref[1, :N, 64:65]
    inv = 1.0 / jnp.maximum(deg, 1.0)
    inv_ref[...] = inv
    agg = (aggp_ref[0, :N, :64] + aggp_ref[1, :N, :64]) * inv
    h2 = jnp.dot(agg, wl_ref[...], preferred_element_type=jnp.float32)
    h2 = h2 + bl_ref[...]
    h2 = h2 + jnp.dot(h1_ref[:, :64], wr_ref[...],
                      preferred_element_type=jnp.float32)
    h2 = _leaky(h2)
    h2_ref[...] = h2
    g2_ref[...] = jnp.dot(h2, w2l_ref[...], preferred_element_type=jnp.float32)

  return pl.pallas_call(
      body,
      out_shape=[
          jax.ShapeDtypeStruct((N, 128), jnp.float32),
          jax.ShapeDtypeStruct((N, 64), jnp.float32),
          jax.ShapeDtypeStruct((N, 1), jnp.float32),
      ],
  )(aggp, h1p, s1wl, s1bl, s1wr, s2wl)


def _tc_final(agg2p, inv, h2, s2bl, s2wr, l2w1, l2b1, l2w2, l2b2, batch_row):
  def body(agg2p_ref, inv_ref, h2_ref, bl_ref, wr_ref, w1_ref, b1_ref,
           w2_ref, b2_ref, batch_ref, o_ref):
    t = (agg2p_ref[0, :N, :] + agg2p_ref[1, :N, :]) * inv_ref[...]
    t = t + bl_ref[...]
    t = t + jnp.dot(h2_ref[...], wr_ref[...],
                    preferred_element_type=jnp.float32)
    t = _leaky(t)
    t = jnp.dot(t, w1_ref[...], preferred_element_type=jnp.float32)
    t = jnp.maximum(t + b1_ref[...], 0.0)
    t = jnp.dot(t, w2_ref[...], preferred_element_type=jnp.float32)
    t = t + b2_ref[...]
    onehot_t = (lax.broadcasted_iota(jnp.int32, (NUM_GRAPHS, N), 0)
                == batch_ref[...]).astype(jnp.float32)
    o_ref[...] = jnp.dot(onehot_t, t, preferred_element_type=jnp.float32)

  return pl.pallas_call(
      body,
      out_shape=jax.ShapeDtypeStruct((NUM_GRAPHS, 10), jnp.float32),
  )(agg2p, inv, h2, s2bl, s2wr, l2w1, l2b1, l2w2, l2b2, batch_row)


def kernel(x, edge_index, edge_weight, batch,
           l1w1, l1b1, l1w2, l1b2,
           s1wl, s1bl, s1wr,
           s2wl, s2bl, s2wr,
           l2w1, l2b1, l2w2, l2b2):
  src_r = edge_index[0].reshape(NCHUNK, W)
  dst_r = edge_index[1].reshape(NCHUNK, W)
  z80 = jnp.zeros((NP, 80), jnp.float32)
  z64 = jnp.zeros((NP, 64), jnp.float32)

  h1p = _tc_mlp1(x, l1w1, l1b1.reshape(1, -1), l1w2, l1b2.reshape(1, -1))
  aggp = _sc_seg_sum(h1p, src_r, dst_r, z80, 80)
  h2, g2, inv = _tc_mid(aggp, h1p, s1wl, s1bl.reshape(1, -1), s1wr, s2wl)
  agg2p = _sc_seg_sum(g2, src_r, dst_r, z64, 64)
  out = _tc_final(agg2p, inv, h2, s2bl.reshape(1, -1), s2wr,
                  l2w1, l2b1.reshape(1, -1), l2w2, l2b2.reshape(1, -1),
                  batch.reshape(1, N).astype(jnp.int32))
  return out
